# baseline (speedup 1.0000x reference)
import jax
import jax.numpy as jnp
from jax import lax
from jax.experimental import pallas as pl

N = 4096
D = 512
C = 128
BLK = 256
ALPHA = 0.2
_NT = (((1,), (1,)), ((), ()))


def _bf16(x):
    return x.astype(jnp.bfloat16)


def _proj_body(x_ref, enc_ref, watt_ref, xproj_ref, wh_ref):
    xp = jnp.dot(x_ref[...], enc_ref[...], preferred_element_type=jnp.float32)
    wh = jnp.dot(_bf16(xp), watt_ref[...], preferred_element_type=jnp.float32)
    xproj_ref[...] = _bf16(xp)
    wh_ref[...] = _bf16(wh)


def _attn_body(a_ref, wh_ref, xproj_ref, w1_ref, z1_ref):
    i = pl.program_id(0)
    wh = wh_ref[...]
    wh_blk = wh_ref[pl.ds(i * BLK, BLK), :]
    a8 = a_ref[...]
    eblk = lax.dot_general(wh_blk, a8, _NT, preferred_element_type=jnp.float32)
    erow = lax.dot_general(a8, wh, _NT, preferred_element_type=jnp.float32)
    e1 = eblk[:, 0:1]
    e2 = erow[1:2, :]
    m2 = jnp.max(e2)
    t = e1 + m2
    m = jnp.where(t >= 0, t, ALPHA * t)
    sa = e1 + e2
    s = jnp.exp(jnp.where(sa >= 0, sa, ALPHA * sa) - m)
    z = jnp.sum(s, axis=1, keepdims=True)
    acc = jnp.dot(_bf16(s), wh, preferred_element_type=jnp.float32)
    x_ent = xproj_ref[...].astype(jnp.float32) * (acc / z)
    z1_ref[...] = _bf16(
        jnp.dot(_bf16(x_ent), w1_ref[...], preferred_element_type=jnp.float32))


def _gcn1_body(adj_ref, z1_ref, b1_ref, w2_ref, z2_ref):
    y1 = jnp.dot(adj_ref[...], z1_ref[...], preferred_element_type=jnp.float32)
    y1 = jnp.maximum(y1 + b1_ref[...], 0.0)
    z2_ref[...] = _bf16(
        jnp.dot(_bf16(y1), w2_ref[...], preferred_element_type=jnp.float32))


def _gcn2_body(adj_ref, z2_ref, b2_ref, w3_ref, z3_ref):
    y2 = jnp.dot(adj_ref[...], z2_ref[...], preferred_element_type=jnp.float32)
    y2 = y2 + b2_ref[...]
    z3_ref[...] = _bf16(
        jnp.dot(_bf16(y2), w3_ref[...], preferred_element_type=jnp.float32))


def _gcn3_body(adj_ref, z3_ref, b3_ref, out_ref):
    y3 = jnp.dot(adj_ref[...], z3_ref[...], preferred_element_type=jnp.float32)
    y3 = y3 + b3_ref[...]
    m = jnp.max(y3, axis=1, keepdims=True)
    s = y3 - m
    lse = jnp.log(jnp.sum(jnp.exp(s), axis=1, keepdims=True))
    out_ref[...] = s - lse


def _row_blocked(d):
    return pl.BlockSpec((BLK, d), lambda i: (i, 0))


def _whole(r, c):
    return pl.BlockSpec((r, c), lambda i: (0, 0))


def kernel(x_org, adj, encoder1, W_att, a_att, gc1_W, gc1_b, gc2_W, gc2_b,
           gc3_W, gc3_b):
    grid = (N // BLK,)
    xb = _bf16(x_org)
    adjb = _bf16(adj)
    a_pair = jnp.zeros((8, D), jnp.bfloat16).at[0:2].set(_bf16(a_att.reshape(2, D)))

    xproj, wh = pl.pallas_call(
        _proj_body,
        grid=grid,
        in_specs=[_row_blocked(D), _whole(D, D), _whole(D, D)],
        out_specs=[_row_blocked(D), _row_blocked(D)],
        out_shape=[jax.ShapeDtypeStruct((N, D), jnp.bfloat16)] * 2,
    )(xb, _bf16(encoder1), _bf16(W_att))

    z1 = pl.pallas_call(
        _attn_body,
        grid=grid,
        in_specs=[_whole(8, D), _whole(N, D), _row_blocked(D), _whole(D, D)],
        out_specs=_row_blocked(D),
        out_shape=jax.ShapeDtypeStruct((N, D), jnp.bfloat16),
    )(a_pair, wh, xproj, _bf16(gc1_W))

    z2 = pl.pallas_call(
        _gcn1_body,
        grid=grid,
        in_specs=[_row_blocked(N), _whole(N, D), _whole(1, D), _whole(D, D)],
        out_specs=_row_blocked(D),
        out_shape=jax.ShapeDtypeStruct((N, D), jnp.bfloat16),
    )(adjb, z1, gc1_b.reshape(1, D), _bf16(gc2_W))

    z3 = pl.pallas_call(
        _gcn2_body,
        grid=grid,
        in_specs=[_row_blocked(N), _whole(N, D), _whole(1, D), _whole(D, C)],
        out_specs=_row_blocked(C),
        out_shape=jax.ShapeDtypeStruct((N, C), jnp.bfloat16),
    )(adjb, z2, gc2_b.reshape(1, D), _bf16(gc3_W))

    out = pl.pallas_call(
        _gcn3_body,
        grid=grid,
        in_specs=[_row_blocked(N), _whole(N, C), _whole(1, C)],
        out_specs=_row_blocked(C),
        out_shape=jax.ShapeDtypeStruct((N, C), jnp.float32),
    )(adjb, z3, gc3_b.reshape(1, C))

    return out
